# baseline (speedup 1.0000x reference)
import functools
import math

import jax
import jax.numpy as jnp
import numpy as np
from jax.experimental import pallas as pl
from jax.experimental.pallas import tpu as pltpu

DIM = 768
HEADS = 12
HEAD_DIM = 64
K_NEIGH = 64
SCALE = 1.0 / math.sqrt(HEAD_DIM)
NEG = -1e30
QB = 256
WIN = 384


@functools.lru_cache(maxsize=None)
def _route_constants(seq_len: int, k: int, depth: int = 8):
    pos = np.arange(seq_len)
    x = pos.astype(np.float32) / np.float32(max(1, seq_len - 1))
    x = np.clip(x, np.float32(1e-06), np.float32(1.0 - 1e-06)).astype(np.float32)
    val = np.zeros_like(x)
    factor = 0.5
    for _ in range(depth):
        x_scaled = x * np.float32(3.0)
        digit = x_scaled.astype(np.int32)
        x_frac = (x_scaled - digit.astype(np.float32)).astype(np.float32)
        val = (val + (digit == 2).astype(np.float32) * np.float32(factor)).astype(np.float32)
        x = x_frac
        factor *= 0.5
    val = np.clip(val, 0.0, 1.0).astype(np.float32)
    dist = np.abs(val[:, None] - val[None, :])
    routes = np.argsort(dist, axis=1, kind="stable")[:, :k]

    perm = np.argsort(val, kind="stable")
    rank = np.empty(seq_len, dtype=np.int64)
    rank[perm] = np.arange(seq_len)

    nbr_ranks = rank[routes]
    nbr_sorted = nbr_ranks[perm]
    lo = nbr_sorted.min(axis=1)
    hi = nbr_sorted.max(axis=1)

    n_blocks = seq_len // QB
    ws = np.zeros(n_blocks, dtype=np.int32)
    bias = np.full((seq_len, WIN), NEG, dtype=np.float32)
    for b in range(n_blocks):
        r0, r1 = b * QB, (b + 1) * QB
        start = (lo[r0:r1].min() // 16) * 16
        start = min(int(start), seq_len - WIN)
        assert hi[r0:r1].max() < start + WIN
        ws[b] = start
        for r in range(r0, r1):
            bias[r, nbr_sorted[r] - start] = 0.0
    return perm.astype(np.int32), rank.astype(np.int32), ws, bias


def _fused_kernel(ws_ref, x_ref, wq_ref, wk_ref, wv_ref, bq_ref, bk_ref,
                  bv_ref, bias_ref, wo_ref, bo_ref, o_ref, k_scr, v_scr):
    h = pl.program_id(0)
    x = x_ref[...]
    q = (jnp.dot(x, wq_ref[0], preferred_element_type=jnp.float32)
         + bq_ref[0]).astype(jnp.bfloat16)
    k_scr[...] = (jnp.dot(x, wk_ref[0], preferred_element_type=jnp.float32)
                  + bk_ref[0]).astype(jnp.bfloat16)
    v_scr[...] = (jnp.dot(x, wv_ref[0], preferred_element_type=jnp.float32)
                  + bv_ref[0]).astype(jnp.bfloat16)

    n_blocks = x.shape[0] // QB
    outs = []
    for b in range(n_blocks):
        ws = pl.multiple_of(ws_ref[b], 16)
        qb = q[b * QB:(b + 1) * QB]
        kw = k_scr[pl.ds(ws, WIN), :]
        vw = v_scr[pl.ds(ws, WIN), :]
        s = (
            jnp.dot(qb, kw.T, preferred_element_type=jnp.float32) * SCALE
            + bias_ref[b * QB:(b + 1) * QB]
        )
        m = jnp.max(s, axis=-1, keepdims=True)
        e = jnp.exp(s - m)
        p = (e / jnp.sum(e, axis=-1, keepdims=True)).astype(jnp.bfloat16)
        outs.append(jnp.dot(p, vw, preferred_element_type=jnp.float32))
    o = jnp.concatenate(outs, axis=0).astype(jnp.bfloat16)
    contrib = jnp.dot(o, wo_ref[0], preferred_element_type=jnp.float32)

    @pl.when(h == 0)
    def _init():
        o_ref[...] = contrib + bo_ref[...]

    @pl.when(h != 0)
    def _acc():
        o_ref[...] = o_ref[...] + contrib


def kernel(x, W_qkv, b_qkv, W_out, b_out):
    B, S, D = x.shape
    H, hd = HEADS, HEAD_DIM
    perm_np, rank_np, ws_np, bias_np = _route_constants(S, K_NEIGH)
    perm = jnp.asarray(perm_np)
    invperm = jnp.asarray(rank_np)
    ws = jnp.asarray(ws_np)
    bias = jnp.asarray(bias_np)

    x_perm = x.reshape(S, D)[perm].astype(jnp.bfloat16)
    w_hm = W_qkv.reshape(3 * H, hd, D).transpose(0, 2, 1).astype(jnp.bfloat16)
    b_hm = b_qkv.reshape(3 * H, 1, hd)
    wo_t = W_out.T.reshape(H, hd, D).astype(jnp.bfloat16)
    bo = b_out.reshape(1, D)

    out = pl.pallas_call(
        _fused_kernel,
        grid=(H,),
        in_specs=[
            pl.BlockSpec(memory_space=pltpu.SMEM),
            pl.BlockSpec((S, D), lambda h: (0, 0)),
            pl.BlockSpec((1, D, hd), lambda h: (h, 0, 0)),
            pl.BlockSpec((1, D, hd), lambda h: (H + h, 0, 0)),
            pl.BlockSpec((1, D, hd), lambda h: (2 * H + h, 0, 0)),
            pl.BlockSpec((1, 1, hd), lambda h: (h, 0, 0)),
            pl.BlockSpec((1, 1, hd), lambda h: (H + h, 0, 0)),
            pl.BlockSpec((1, 1, hd), lambda h: (2 * H + h, 0, 0)),
            pl.BlockSpec((S, WIN), lambda h: (0, 0)),
            pl.BlockSpec((1, hd, D), lambda h: (h, 0, 0)),
            pl.BlockSpec((1, D), lambda h: (0, 0)),
        ],
        out_specs=pl.BlockSpec((S, D), lambda h: (0, 0)),
        out_shape=jax.ShapeDtypeStruct((S, D), jnp.float32),
        scratch_shapes=[
            pltpu.VMEM((S, hd), jnp.bfloat16),
            pltpu.VMEM((S, hd), jnp.bfloat16),
        ],
    )(ws, x_perm, w_hm, w_hm, w_hm, b_hm, b_hm, b_hm, bias, wo_t, bo)

    return out[invperm].reshape(B, S, D)


# speedup vs baseline: 1.1466x; 1.1466x over previous
import functools
import math

import jax
import jax.numpy as jnp
import numpy as np
from jax.experimental import pallas as pl
from jax.experimental.pallas import tpu as pltpu

DIM = 768
HEADS = 12
HEAD_DIM = 64
K_NEIGH = 64
SCALE = 1.0 / math.sqrt(HEAD_DIM)
NEG = -1e30
QB = 256
WIN = 384

_CONTRACT_LAST = (((1,), (1,)), ((), ()))


@functools.lru_cache(maxsize=None)
def _route_constants(seq_len: int, k: int, depth: int = 8):
    pos = np.arange(seq_len)
    x = pos.astype(np.float32) / np.float32(max(1, seq_len - 1))
    x = np.clip(x, np.float32(1e-06), np.float32(1.0 - 1e-06)).astype(np.float32)
    val = np.zeros_like(x)
    factor = 0.5
    for _ in range(depth):
        x_scaled = x * np.float32(3.0)
        digit = x_scaled.astype(np.int32)
        x_frac = (x_scaled - digit.astype(np.float32)).astype(np.float32)
        val = (val + (digit == 2).astype(np.float32) * np.float32(factor)).astype(np.float32)
        x = x_frac
        factor *= 0.5
    val = np.clip(val, 0.0, 1.0).astype(np.float32)
    dist = np.abs(val[:, None] - val[None, :])
    routes = np.argsort(dist, axis=1, kind="stable")[:, :k]

    perm = np.argsort(val, kind="stable")
    rank = np.empty(seq_len, dtype=np.int64)
    rank[perm] = np.arange(seq_len)

    nbr_ranks = rank[routes]
    nbr_sorted = nbr_ranks[perm]
    lo = nbr_sorted.min(axis=1)
    hi = nbr_sorted.max(axis=1)

    n_blocks = seq_len // QB
    ws = np.zeros(n_blocks, dtype=np.int32)
    bias = np.full((seq_len, WIN), NEG, dtype=np.float32)
    for b in range(n_blocks):
        r0, r1 = b * QB, (b + 1) * QB
        start = (lo[r0:r1].min() // 16) * 16
        start = min(int(start), seq_len - WIN)
        assert hi[r0:r1].max() < start + WIN
        ws[b] = start
        for r in range(r0, r1):
            bias[r, nbr_sorted[r] - start] = 0.0
    return perm.astype(np.int32), rank.astype(np.int32), ws, bias


def _fused_kernel(ws_ref, x_ref, wq_ref, wk_ref, wv_ref, bq_ref, bk_ref,
                  bv_ref, bias_ref, wo_ref, bo_ref, o_ref, k_scr, v_scr):
    h = pl.program_id(0)
    x = x_ref[...]
    q = jax.lax.dot_general(x, wq_ref[0], _CONTRACT_LAST,
                            preferred_element_type=jnp.float32) + bq_ref[0]
    k_scr[...] = jax.lax.dot_general(x, wk_ref[0], _CONTRACT_LAST,
                                     preferred_element_type=jnp.float32) + bk_ref[0]
    v_scr[...] = jax.lax.dot_general(x, wv_ref[0], _CONTRACT_LAST,
                                     preferred_element_type=jnp.float32) + bv_ref[0]

    n_blocks = x.shape[0] // QB
    outs = []
    for b in range(n_blocks):
        ws = pl.multiple_of(ws_ref[b], 16)
        qb = q[b * QB:(b + 1) * QB]
        kw = k_scr[pl.ds(ws, WIN), :]
        vw = v_scr[pl.ds(ws, WIN), :]
        s = (
            jnp.dot(qb, kw.T, preferred_element_type=jnp.float32) * SCALE
            + bias_ref[b * QB:(b + 1) * QB]
        )
        m = jnp.max(s, axis=-1, keepdims=True)
        e = jnp.exp(s - m)
        p = e / jnp.sum(e, axis=-1, keepdims=True)
        outs.append(jnp.dot(p, vw, preferred_element_type=jnp.float32))
    o = jnp.concatenate(outs, axis=0)
    contrib = jnp.dot(o, wo_ref[0], preferred_element_type=jnp.float32)

    @pl.when(h == 0)
    def _init():
        o_ref[...] = contrib + bo_ref[...]

    @pl.when(h != 0)
    def _acc():
        o_ref[...] = o_ref[...] + contrib


def kernel(x, W_qkv, b_qkv, W_out, b_out):
    B, S, D = x.shape
    H, hd = HEADS, HEAD_DIM
    perm_np, rank_np, ws_np, bias_np = _route_constants(S, K_NEIGH)
    perm = jnp.asarray(perm_np)
    invperm = jnp.asarray(rank_np)
    ws = jnp.asarray(ws_np)
    bias = jnp.asarray(bias_np)

    x_perm = x.reshape(S, D)[perm]
    w_hm = W_qkv.reshape(3 * H, hd, D)
    b_hm = b_qkv.reshape(3 * H, 1, hd)
    wo_t = W_out.T.reshape(H, hd, D)
    bo = b_out.reshape(1, D)

    out = pl.pallas_call(
        _fused_kernel,
        grid=(H,),
        in_specs=[
            pl.BlockSpec(memory_space=pltpu.SMEM),
            pl.BlockSpec((S, D), lambda h: (0, 0)),
            pl.BlockSpec((1, hd, D), lambda h: (h, 0, 0)),
            pl.BlockSpec((1, hd, D), lambda h: (H + h, 0, 0)),
            pl.BlockSpec((1, hd, D), lambda h: (2 * H + h, 0, 0)),
            pl.BlockSpec((1, 1, hd), lambda h: (h, 0, 0)),
            pl.BlockSpec((1, 1, hd), lambda h: (H + h, 0, 0)),
            pl.BlockSpec((1, 1, hd), lambda h: (2 * H + h, 0, 0)),
            pl.BlockSpec((S, WIN), lambda h: (0, 0)),
            pl.BlockSpec((1, hd, D), lambda h: (h, 0, 0)),
            pl.BlockSpec((1, D), lambda h: (0, 0)),
        ],
        out_specs=pl.BlockSpec((S, D), lambda h: (0, 0)),
        out_shape=jax.ShapeDtypeStruct((S, D), jnp.float32),
        scratch_shapes=[
            pltpu.VMEM((S, hd), jnp.float32),
            pltpu.VMEM((S, hd), jnp.float32),
        ],
    )(ws, x_perm, w_hm, w_hm, w_hm, b_hm, b_hm, b_hm, bias, wo_t, bo)

    return out[invperm].reshape(B, S, D)


# speedup vs baseline: 1.6496x; 1.4386x over previous
import functools
import math

import jax
import jax.numpy as jnp
import numpy as np
from jax.experimental import pallas as pl
from jax.experimental.pallas import tpu as pltpu

DIM = 768
HEADS = 12
HEAD_DIM = 64
K_NEIGH = 64
SCALE = 1.0 / math.sqrt(HEAD_DIM)
NEG = -1e30
QB = 256
WIN = 384

_CONTRACT_LAST = (((1,), (1,)), ((), ()))


@functools.lru_cache(maxsize=None)
def _route_constants(seq_len: int, k: int, depth: int = 8):
    pos = np.arange(seq_len)
    x = pos.astype(np.float32) / np.float32(max(1, seq_len - 1))
    x = np.clip(x, np.float32(1e-06), np.float32(1.0 - 1e-06)).astype(np.float32)
    val = np.zeros_like(x)
    factor = 0.5
    for _ in range(depth):
        x_scaled = x * np.float32(3.0)
        digit = x_scaled.astype(np.int32)
        x_frac = (x_scaled - digit.astype(np.float32)).astype(np.float32)
        val = (val + (digit == 2).astype(np.float32) * np.float32(factor)).astype(np.float32)
        x = x_frac
        factor *= 0.5
    val = np.clip(val, 0.0, 1.0).astype(np.float32)
    dist = np.abs(val[:, None] - val[None, :])
    routes = np.argsort(dist, axis=1, kind="stable")[:, :k]

    perm = np.argsort(val, kind="stable")
    rank = np.empty(seq_len, dtype=np.int64)
    rank[perm] = np.arange(seq_len)

    nbr_ranks = rank[routes]
    nbr_sorted = nbr_ranks[perm]
    lo = nbr_sorted.min(axis=1)
    hi = nbr_sorted.max(axis=1)

    n_blocks = seq_len // QB
    ws = np.zeros(n_blocks, dtype=np.int32)
    bias = np.full((seq_len, WIN), NEG, dtype=np.float32)
    for b in range(n_blocks):
        r0, r1 = b * QB, (b + 1) * QB
        start = (lo[r0:r1].min() // 16) * 16
        start = min(int(start), seq_len - WIN)
        assert hi[r0:r1].max() < start + WIN
        ws[b] = start
        for r in range(r0, r1):
            bias[r, nbr_sorted[r] - start] = 0.0
    return perm.astype(np.int32), rank.astype(np.int32), ws, bias


def _fused_kernel(ws_ref, x_ref, w_ref, b_ref, wo_ref, bo_ref, bias_ref,
                  o_ref, k_scr, v_scr, oh_scr):
    b = pl.program_id(0)

    @pl.when(b == 0)
    def _prime_kv():
        x = x_ref[...]
        k_scr[...] = jax.lax.dot_general(
            x, w_ref[DIM:2 * DIM], _CONTRACT_LAST,
            preferred_element_type=jnp.float32) + b_ref[0, DIM:2 * DIM]
        v_scr[...] = jax.lax.dot_general(
            x, w_ref[2 * DIM:], _CONTRACT_LAST,
            preferred_element_type=jnp.float32) + b_ref[0, 2 * DIM:]

    ws = pl.multiple_of(ws_ref[b], 16)
    kw = k_scr[pl.ds(ws, WIN), :]
    vw = v_scr[pl.ds(ws, WIN), :]
    bias_b = bias_ref[...]
    qb_all = (
        jax.lax.dot_general(x_ref[pl.ds(b * QB, QB)], w_ref[:DIM],
                            _CONTRACT_LAST,
                            preferred_element_type=jnp.float32)
        + b_ref[0, :DIM]
    ) * SCALE
    for h in range(HEADS):
        c0, c1 = h * HEAD_DIM, (h + 1) * HEAD_DIM
        qb = qb_all[:, c0:c1]
        s = (
            jnp.dot(qb, kw[:, c0:c1].T, preferred_element_type=jnp.float32)
            + bias_b
        )
        m = jnp.max(s, axis=-1, keepdims=True)
        e = jnp.exp(s - m)
        o_hb = jnp.dot(e, vw[:, c0:c1], preferred_element_type=jnp.float32)
        o_hb = o_hb / jnp.sum(e, axis=-1, keepdims=True)
        oh_scr[:, c0:c1] = o_hb
    o_ref[...] = (
        jax.lax.dot_general(oh_scr[...], wo_ref[...], _CONTRACT_LAST,
                            preferred_element_type=jnp.float32)
        + bo_ref[...]
    )


def kernel(x, W_qkv, b_qkv, W_out, b_out):
    B, S, D = x.shape
    H, hd = HEADS, HEAD_DIM
    perm_np, rank_np, ws_np, bias_np = _route_constants(S, K_NEIGH)
    perm = jnp.asarray(perm_np)
    invperm = jnp.asarray(rank_np)
    ws = jnp.asarray(ws_np)
    bias = jnp.asarray(bias_np)

    x_perm = x.reshape(S, D)[perm]

    out = pl.pallas_call(
        _fused_kernel,
        grid=(S // QB,),
        in_specs=[
            pl.BlockSpec(memory_space=pltpu.SMEM),
            pl.BlockSpec((S, D), lambda b: (0, 0)),
            pl.BlockSpec((3 * D, D), lambda b: (0, 0)),
            pl.BlockSpec((1, 3 * D), lambda b: (0, 0)),
            pl.BlockSpec((D, D), lambda b: (0, 0)),
            pl.BlockSpec((1, D), lambda b: (0, 0)),
            pl.BlockSpec((QB, WIN), lambda b: (b, 0)),
        ],
        out_specs=pl.BlockSpec((QB, D), lambda b: (b, 0)),
        out_shape=jax.ShapeDtypeStruct((S, D), jnp.float32),
        scratch_shapes=[
            pltpu.VMEM((S, D), jnp.float32),
            pltpu.VMEM((S, D), jnp.float32),
            pltpu.VMEM((QB, D), jnp.float32),
        ],
    )(ws, x_perm, W_qkv, b_qkv.reshape(1, 3 * D), W_out, b_out.reshape(1, D),
      bias)

    return out[invperm].reshape(B, S, D)
